# baseline (speedup 1.0000x reference)
import jax
import jax.numpy as jnp
from jax import lax
from jax.experimental import pallas as pl
from jax.experimental.pallas import tpu as pltpu
from jax.experimental.pallas import tpu_sc as plsc

H = 256
E = 60000
NC = 2
NS = 16
NW = NC * NS
IDXB = 1024
E_PAD = -(-E // (2 * IDXB)) * (2 * IDXB)
NBLK = E_PAD // IDXB
GB = 32
RING = 256
ACC_R = 320

ETYPE_INFO = [
    ("txm", 0, 2000, 64),
    ("ud", 0, 4000, 128),
    ("sb", 1, 10000, 320),
    ("ps", 0, 10000, 320),
    ("eft", 0, 10000, 320),
]


def _pad_edges(ei):
    src = jnp.concatenate([ei[0], jnp.zeros((E_PAD - E,), jnp.int32)])
    dst = jnp.concatenate([ei[1], jnp.full((E_PAD - E,), -1, jnp.int32)])
    return src, dst


def _sc_agg_body(x_acc, x_dev, s_txm, d_txm, s_ud, d_ud, s_sb, d_sb, s_ps,
                 d_ps, s_eft, d_eft, o_txm, o_ud, o_sb, o_ps, o_eft,
                 sidx_a, sidx_b, didx_a, didx_b, rows, acc, cnt, stg_s, stg_d,
                 gsem, isem_a, isem_b):
    c = lax.axis_index("c")
    s = lax.axis_index("s")
    wid = s * NC + c
    xs = [x_acc, x_dev]
    srcs = [s_txm, s_ud, s_sb, s_ps, s_eft]
    dsts = [d_txm, d_ud, d_sb, d_ps, d_eft]
    outs = [o_txm, o_ud, o_sb, o_ps, o_eft]
    ones16 = jnp.full((16,), 1.0, jnp.float32)
    zero16f = jnp.zeros((16,), jnp.float32)
    zero16i = jnp.zeros((16,), jnp.int32)
    neg16i = jnp.full((16,), -1, jnp.int32)

    for t, (_, xi, n_dst, rng) in enumerate(ETYPE_INFO):
        x_src = xs[xi]
        esrc, edst = srcs[t], dsts[t]
        base = wid * rng
        lastr = n_dst - (NW - 1) * rng

        def zr(i, _):
            for k in range(H // 16):
                acc[i, pl.ds(k * 16, 16)] = zero16f
            return 0

        lax.fori_loop(0, rng, zr, 0)
        for k in range(rng // 16):
            cnt[pl.ds(k * 16, 16)] = zero16f

        iota16 = lax.iota(jnp.int32, 16)
        lane0 = iota16 == 0

        def fire(a_r):
            rp = pl.multiple_of(a_r & (RING - 1), GB)
            pltpu.async_copy(x_src.at[stg_s.at[pl.ds(rp, GB)]], rows,
                             gsem).wait()

            for q in range(GB // 16):
                lv16 = stg_d[pl.ds(rp + q * 16, 16)]

                def acc_e(rr, _):
                    d = jnp.sum(jnp.where(iota16 == rr, lv16, 0))

                    @pl.when(d >= 0)
                    def _():
                        for k in range(H // 16):
                            plsc.addupdate(acc.at[d, pl.ds(k * 16, 16)],
                                           rows[q * 16 + rr,
                                                pl.ds(k * 16, 16)])
                        plsc.addupdate_scatter(
                            cnt, [jnp.full((16,), d, jnp.int32)], ones16,
                            mask=lane0)
                    return 0

                lax.fori_loop(0, 16, acc_e, 0)
            return a_r + GB

        def maybe_fire(a_w, a_r):
            return lax.cond(a_w - a_r >= GB, fire, lambda r: r, a_r)

        def chunk(sref, dref, j, a_w, a_r):
            dval = dref[pl.ds(j * 16, 16)]
            sval = sref[pl.ds(j * 16, 16)]
            lv = dval - base
            ok = (lv >= 0) & (lv < rng)
            cum = plsc.cumsum(ok.astype(jnp.int32))
            wloc = (a_w + cum - 1) & (RING - 1)
            plsc.store_scatter(stg_s, [wloc], sval, mask=ok)
            plsc.store_scatter(stg_d, [wloc], lv, mask=ok)
            a_w = a_w + jnp.max(cum)
            return a_w, maybe_fire(a_w, a_r)

        def blkproc(sref, dref, carry):
            def ch(j, cr):
                return chunk(sref, dref, j, *cr)
            return lax.fori_loop(0, IDXB // 16, ch, carry)

        def pf_start(blk, sbuf, dbuf, sem):
            off = pl.multiple_of(blk * IDXB, IDXB)
            pltpu.make_async_copy(esrc.at[pl.ds(off, IDXB)], sbuf, sem).start()
            pltpu.make_async_copy(edst.at[pl.ds(off, IDXB)], dbuf, sem).start()

        def pf_wait(sbuf, dbuf, sem):
            pltpu.make_async_copy(esrc.at[pl.ds(0, IDXB)], sbuf, sem).wait()
            pltpu.make_async_copy(edst.at[pl.ds(0, IDXB)], dbuf, sem).wait()

        pf_start(0, sidx_a, didx_a, isem_a)
        pf_start(1, sidx_b, didx_b, isem_b)

        npair = NBLK // 2

        def outer(g, carry):
            pf_wait(sidx_a, didx_a, isem_a)
            carry = blkproc(sidx_a, didx_a, carry)

            @pl.when(g < npair - 1)
            def _():
                pf_start(2 * g + 2, sidx_a, didx_a, isem_a)

            pf_wait(sidx_b, didx_b, isem_b)
            carry = blkproc(sidx_b, didx_b, carry)

            @pl.when(g < npair - 1)
            def _():
                pf_start(2 * g + 3, sidx_b, didx_b, isem_b)
            return carry

        a_w, a_r = lax.fori_loop(0, npair, outer, (0, 0))

        for _ in range(GB // 16):
            wloc = (a_w + iota16) & (RING - 1)
            plsc.store_scatter(stg_s, [wloc], zero16i)
            plsc.store_scatter(stg_d, [wloc], neg16i)
            a_w = a_w + 16
        a_r = maybe_fire(a_w, a_r)
        a_r = maybe_fire(a_w, a_r)
        a_r = maybe_fire(a_w, a_r)

        def divrows(r, _):
            c16 = cnt[pl.ds(r * 16, 16)]
            inv16 = 1.0 / jnp.maximum(c16, 1.0)

            def divrow(rr, _):
                iv = jnp.sum(jnp.where(iota16 == rr, inv16, 0.0))
                d = r * 16 + rr
                for k in range(H // 16):
                    acc[d, pl.ds(k * 16, 16)] = acc[d, pl.ds(k * 16, 16)] * iv
                return 0

            lax.fori_loop(0, 16, divrow, 0)
            return 0

        lax.fori_loop(0, rng // 16, divrows, 0)

        def drain(rows_n):
            dst0 = pl.multiple_of(wid * rng, 8)
            pltpu.sync_copy(acc.at[pl.ds(0, rows_n), pl.ds(0, H)],
                            outs[t].at[pl.ds(dst0, rows_n)])

        @pl.when(wid < NW - 1)
        def _():
            drain(rng)

        @pl.when(wid == NW - 1)
        def _():
            drain(lastr)


def _sc_aggregate(x_acc, x_dev, edges):
    out_type = [jax.ShapeDtypeStruct((n, H), jnp.float32)
                for (_, _, n, _) in ETYPE_INFO]
    scratch = [
        pltpu.VMEM((IDXB,), jnp.int32),
        pltpu.VMEM((IDXB,), jnp.int32),
        pltpu.VMEM((IDXB,), jnp.int32),
        pltpu.VMEM((IDXB,), jnp.int32),
        pltpu.VMEM((GB, H), jnp.float32),
        pltpu.VMEM((ACC_R, H), jnp.float32),
        pltpu.VMEM((ACC_R,), jnp.float32),
        pltpu.VMEM((RING,), jnp.int32),
        pltpu.VMEM((RING,), jnp.int32),
        pltpu.SemaphoreType.DMA,
        pltpu.SemaphoreType.DMA,
        pltpu.SemaphoreType.DMA,
    ]
    mesh = plsc.VectorSubcoreMesh(core_axis_name="c", subcore_axis_name="s")
    fn = pl.kernel(_sc_agg_body, out_type=out_type, mesh=mesh,
                   scratch_types=scratch,
                   compiler_params=pltpu.CompilerParams(
                       needs_layout_passes=False))
    flat = []
    for sd in edges:
        flat.extend(sd)
    return fn(x_acc, x_dev, *flat)


def _proj_body(xa, xd, xm, wa, ba, wd, bd, wm, bm, oa, od, om):
    oa[...] = jnp.dot(xa[...], wa[...], preferred_element_type=jnp.float32) + ba[...][None, :]
    od[...] = jnp.dot(xd[...], wd[...], preferred_element_type=jnp.float32) + bd[...][None, :]
    om[...] = jnp.dot(xm[...], wm[...], preferred_element_type=jnp.float32) + bm[...][None, :]


def _bn_relu(pre, g, b):
    m = jnp.mean(pre, axis=0, keepdims=True)
    v = jnp.mean((pre - m) ** 2, axis=0, keepdims=True)
    y = (pre - m) * lax.rsqrt(v + 1e-5) * g[None, :] + b[None, :]
    return jnp.maximum(y, 0.0)


def _dense_one_body(m_ref, x_ref, wl, bl, wr, g, b, o_ref):
    pre = (jnp.dot(m_ref[...], wl[...], preferred_element_type=jnp.float32)
           + jnp.dot(x_ref[...], wr[...], preferred_element_type=jnp.float32)
           + bl[...][None, :])
    o_ref[...] = _bn_relu(pre, g[...], b[...])


def _dense_acc_pre_body(m1, m2, m3, x_ref, wl1, bl1, wl2, bl2, wl3,
                        bl3, wr1, wr2, wr3, o_ref):
    o_ref[...] = (jnp.dot(m1[...], wl1[...],
                          preferred_element_type=jnp.float32)
                  + jnp.dot(m2[...], wl2[...],
                            preferred_element_type=jnp.float32)
                  + jnp.dot(m3[...], wl3[...],
                            preferred_element_type=jnp.float32)
                  + jnp.dot(x_ref[...], wr1[...] + wr2[...] + wr3[...],
                            preferred_element_type=jnp.float32)
                  + (bl1[...] + bl2[...] + bl3[...])[None, :])


def _bn_relu_body(x_ref, g, b, o_ref):
    o_ref[...] = _bn_relu(x_ref[...], g[...], b[...])


def _dense_acc(m1, m2, m3, x, wl1, bl1, wl2, bl2, wl3, bl3, wr1, wr2, wr3,
               g, b):
    n = x.shape[0]
    blk = 2000
    row_spec = pl.BlockSpec((blk, H), lambda i: (i, 0))
    w_spec = pl.BlockSpec((H, H), lambda i: (0, 0))
    b_spec = pl.BlockSpec((H,), lambda i: (0,))
    pre = pl.pallas_call(
        _dense_acc_pre_body,
        grid=(n // blk,),
        in_specs=[row_spec, row_spec, row_spec, row_spec,
                  w_spec, b_spec, w_spec, b_spec, w_spec, b_spec,
                  w_spec, w_spec, w_spec],
        out_specs=row_spec,
        out_shape=jax.ShapeDtypeStruct((n, H), jnp.float32),
    )(m1, m2, m3, x, wl1, bl1, wl2, bl2, wl3, bl3, wr1, wr2, wr3)
    return _tc_call(_bn_relu_body, jax.ShapeDtypeStruct((n, H), jnp.float32),
                    pre, g, b)


def _cls_body(x_ref, w1, b1, w2, b2, o_ref):
    h = jnp.maximum(
        jnp.dot(x_ref[...], w1[...], preferred_element_type=jnp.float32)
        + b1[...][None, :], 0.0)
    o_ref[...] = jnp.sum(h * w2[...][:, 0][None, :], axis=1) + b2[...]


def _tc_call(body, out_shape, *args):
    return pl.pallas_call(body, out_shape=out_shape)(*args)


def kernel(x_account, x_device, x_merchant, ei_txm, ei_ud, ei_sb, ei_ps,
           ei_eft, params):
    p = params
    edges = [_pad_edges(ei) for ei in (ei_txm, ei_ud, ei_sb, ei_ps, ei_eft)]

    acc, dev, mer = _tc_call(
        _proj_body,
        [jax.ShapeDtypeStruct((10000, H), jnp.float32),
         jax.ShapeDtypeStruct((4000, H), jnp.float32),
         jax.ShapeDtypeStruct((2000, H), jnp.float32)],
        x_account, x_device, x_merchant,
        p['proj_acc_W'], p['proj_acc_b'], p['proj_dev_W'], p['proj_dev_b'],
        p['proj_mer_W'], p['proj_mer_b'])

    for l in range(3):
        s_txm, s_ud, s_sb, s_ps, s_eft = _sc_aggregate(acc, dev, edges)

        new_mer = _tc_call(
            _dense_one_body, jax.ShapeDtypeStruct((2000, H), jnp.float32),
            s_txm, mer,
            p['l%d_txm_Wl' % l], p['l%d_txm_bl' % l], p['l%d_txm_Wr' % l],
            p['l%d_bn_mer_g' % l], p['l%d_bn_mer_b' % l])
        new_dev = _tc_call(
            _dense_one_body, jax.ShapeDtypeStruct((4000, H), jnp.float32),
            s_ud, dev,
            p['l%d_ud_Wl' % l], p['l%d_ud_bl' % l], p['l%d_ud_Wr' % l],
            p['l%d_bn_dev_g' % l], p['l%d_bn_dev_b' % l])
        new_acc = _dense_acc(
            s_sb, s_ps, s_eft, acc,
            p['l%d_sb_Wl' % l], p['l%d_sb_bl' % l],
            p['l%d_ps_Wl' % l], p['l%d_ps_bl' % l],
            p['l%d_eft_Wl' % l], p['l%d_eft_bl' % l],
            p['l%d_sb_Wr' % l], p['l%d_ps_Wr' % l], p['l%d_eft_Wr' % l],
            p['l%d_bn_acc_g' % l], p['l%d_bn_acc_b' % l])
        acc, dev, mer = new_acc, new_dev, new_mer

    return _tc_call(_cls_body, jax.ShapeDtypeStruct((10000,), jnp.float32),
                    acc, p['cls_W1'], p['cls_b1'], p['cls_W2'], p['cls_b2'])
